# baseline (speedup 1.0000x reference)
import functools

import jax
import jax.numpy as jnp
from jax import lax
from jax.experimental import pallas as pl
from jax.experimental.pallas import tpu as pltpu
from jax.experimental.pallas import tpu_sc as plsc

LANES = 16
STAGE = 256
SHIFT = 8
RING = 64
NW = 32


def _geom(n):
    aligned = (n // 128) * 128
    n_full = aligned // STAGE
    last_dma = aligned - n_full * STAGE
    tail = n - aligned
    n_stages = n_full + (1 if (last_dma or tail) else 0)
    return n_full, last_dma, tail, n_stages


def _scan_kernel(batch, nf, n_user, n_item):
    uf, ul, utl, us = _geom(n_user)
    itf, il, itl, is_ = _geom(n_item)
    max_it_u = (us + NW - 1) // NW
    max_it_i = (is_ + NW - 1) // NW

    mesh = plsc.VectorSubcoreMesh(core_axis_name="c", subcore_axis_name="s")

    @functools.partial(
        pl.kernel,
        mesh=mesh,
        out_type=(jax.ShapeDtypeStruct((batch, nf), jnp.float32),
                  jax.ShapeDtypeStruct((batch, nf), jnp.float32)),
        compiler_params=pltpu.CompilerParams(needs_layout_passes=False),
        scratch_types=[
            pltpu.VMEM((batch,), jnp.int32),
            pltpu.VMEM((batch,), jnp.int32),
            pltpu.VMEM((2, nf, STAGE), jnp.float32),
            pltpu.VMEM((RING, nf), jnp.float32),
            pltpu.VMEM((64 * 64,), jnp.float32),
            pltpu.SemaphoreType.DMA,
            pltpu.SemaphoreType.DMA,
        ],
    )
    def sc_kernel(u_hbm, i_hbm, uvT_hbm, ivT_hbm, utail_hbm, itail_hbm,
                  uout_hbm, iout_hbm,
                  idx_v, hits_v, blk, ring, tail_v, semb, semo):
        wid = lax.axis_index("s") * 2 + lax.axis_index("c")
        lane_iota = lax.broadcasted_iota(jnp.int32, (LANES,), 0)
        zeros16 = jnp.zeros((LANES,), jnp.int32)

        def extract(vec, k):
            return jnp.sum(jnp.where(lane_iota == k, vec, 0))

        def lag_wait(out_hbm):
            pltpu.make_async_copy(ring.at[pl.ds(0, 1), :],
                                  out_hbm.at[pl.ds(0, 1), :], semo).wait()

        def run_phase(src_idx_hbm, table_hbm, tail_hbm, out_hbm,
                      n_full, last_dma, tail, n_stages, max_iters, rc0):
            pltpu.sync_copy(src_idx_hbm, idx_v)
            if tail:
                pltpu.sync_copy(tail_hbm, tail_v.at[pl.ds(0, tail * nf)])

            def comp(g, cnt):
                v = idx_v[pl.ds(g * LANES, LANES)]
                mine = ((v >> SHIFT) & (NW - 1)) == wid
                mi = mine.astype(jnp.int32)
                off = plsc.cumsum(mi) - 1
                plsc.store_scatter(hits_v, [cnt + off],
                                   g * LANES + lane_iota, mask=mine)
                return cnt + jnp.sum(mi)

            nhits = lax.fori_loop(0, batch // LANES, comp, 0)
            nhg = (nhits + LANES - 1) // LANES
            last_stage = n_stages - 1

            def fire(s, p):
                @pl.when(s < n_full)
                def _():
                    pltpu.async_copy(
                        table_hbm.at[:, pl.ds(s * STAGE, STAGE)],
                        blk.at[p], semb)
                if last_dma:
                    @pl.when(s == n_full)
                    def _():
                        pltpu.async_copy(
                            table_hbm.at[:, pl.ds(n_full * STAGE, last_dma)],
                            blk.at[p, :, pl.ds(0, last_dma)], semb)

            def wait_blk(s):
                @pl.when(s < n_full)
                def _():
                    pltpu.make_async_copy(
                        table_hbm.at[:, pl.ds(0, STAGE)],
                        blk.at[0], semb).wait()
                if last_dma:
                    @pl.when(s == n_full)
                    def _():
                        pltpu.make_async_copy(
                            table_hbm.at[:, pl.ds(0, last_dma)],
                            blk.at[0, :, pl.ds(0, last_dma)], semb).wait()

            def backfill(p, col0):
                for c in range(tail):
                    for t in range(nf // LANES):
                        flat = (t * LANES + lane_iota) * tail + c
                        vals = plsc.load_gather(tail_v, [flat])
                        plsc.store_scatter(
                            blk, [zeros16 + p, t * LANES + lane_iota,
                                  zeros16 + (col0 + c)], vals)

            fire(wid, 0)

            def stage_body(k, rcnt):
                s = wid + k * NW

                def process(rc):
                    p = k & 1
                    nxt = s + NW

                    @pl.when(nxt < n_stages)
                    def _():
                        fire(nxt, (k + 1) & 1)

                    wait_blk(s)
                    if tail:
                        @pl.when(s == last_stage)
                        def _():
                            backfill(p, last_dma)

                    base = s * STAGE

                    def hit_group(g, rc):
                        hb = hits_v[pl.ds(g * LANES, LANES)] & (batch - 1)
                        hu = plsc.load_gather(idx_v, [hb])
                        valid = (g * LANES + lane_iota) < nhits
                        smask = ((hu >> SHIFT) == s) & valid
                        smi = smask.astype(jnp.int32)
                        nmatch = jnp.sum(smi)

                        def do_lanes(rc):
                            def lane(kk, rc):
                                def work(rc):
                                    b = extract(hb, kk)
                                    col = extract(hu, kk) - base

                                    @pl.when(rc >= RING)
                                    def _():
                                        lag_wait(out_hbm)
                                    slot = rc % RING
                                    for t in range(nf // LANES):
                                        vals = plsc.load_gather(
                                            blk,
                                            [zeros16 + p,
                                             t * LANES + lane_iota,
                                             zeros16 + col])
                                        ring[slot,
                                             pl.ds(t * LANES, LANES)] = vals
                                    pltpu.async_copy(
                                        ring.at[pl.ds(slot, 1), :],
                                        out_hbm.at[pl.ds(b, 1), :], semo)
                                    return rc + 1

                                mk = extract(smi, kk) > 0
                                return lax.cond(mk, work, lambda r: r, rc)

                            return lax.fori_loop(0, LANES, lane, rc)

                        return lax.cond(nmatch > 0, do_lanes,
                                        lambda r: r, rc)

                    return lax.fori_loop(0, nhg, hit_group, rc)

                return lax.cond(s < n_stages, process,
                                lambda r: r, rcnt)

            return lax.fori_loop(0, max_iters, stage_body, rc0)

        rc = run_phase(u_hbm, uvT_hbm, utail_hbm, uout_hbm,
                       uf, ul, utl, us, max_it_u, 0)
        rc = run_phase(i_hbm, ivT_hbm, itail_hbm, iout_hbm,
                       itf, il, itl, is_, max_it_i, rc)

        def drain(j, _):
            @pl.when(j < jnp.minimum(rc, RING))
            def _():
                lag_wait(uout_hbm)
            return _

        lax.fori_loop(0, RING, drain, None)

    return sc_kernel


def _dot_kernel(batch, nf):
    blk = 1024

    def body(a_ref, b_ref, o_ref):
        o_ref[...] = jnp.sum(a_ref[...] * b_ref[...], axis=1)

    return pl.pallas_call(
        body,
        grid=(batch // blk,),
        in_specs=[pl.BlockSpec((blk, nf), lambda g: (g, 0)),
                  pl.BlockSpec((blk, nf), lambda g: (g, 0))],
        out_specs=pl.BlockSpec((blk,), lambda g: (g,)),
        out_shape=jax.ShapeDtypeStruct((batch,), jnp.float32),
    )


def kernel(u, i, user_vec, item_vec):
    batch = u.shape[0]
    n_user, nf = user_vec.shape
    n_item = item_vec.shape[0]
    uvT = user_vec.T
    ivT = item_vec.T
    utail = uvT[:, (n_user // 128) * 128:].reshape(-1)
    itail = ivT[:, (n_item // 128) * 128:].reshape(-1)
    scan = _scan_kernel(batch, nf, n_user, n_item)
    uvg, ivg = scan(u, i, uvT, ivT, utail, itail)
    return _dot_kernel(batch, nf)(uvg, ivg)


# speedup vs baseline: 1.0068x; 1.0068x over previous
import functools

import jax
import jax.numpy as jnp
from jax import lax
from jax.experimental import pallas as pl
from jax.experimental.pallas import tpu as pltpu
from jax.experimental.pallas import tpu_sc as plsc

LANES = 16
STAGE = 256
SHIFT = 8
RING = 64
NW = 32


def _geom(n):
    aligned = (n // 128) * 128
    n_full = aligned // STAGE
    last_dma = aligned - n_full * STAGE
    tail = n - aligned
    n_stages = n_full + (1 if (last_dma or tail) else 0)
    return n_full, last_dma, tail, n_stages


def _scan_kernel(batch, nf, n_user, n_item):
    uf, ul, utl, us = _geom(n_user)
    itf, il, itl, is_ = _geom(n_item)
    max_it_u = (us + NW - 1) // NW
    max_it_i = (is_ + NW - 1) // NW

    mesh = plsc.VectorSubcoreMesh(core_axis_name="c", subcore_axis_name="s")

    @functools.partial(
        pl.kernel,
        mesh=mesh,
        out_type=(jax.ShapeDtypeStruct((batch, nf), jnp.float32),
                  jax.ShapeDtypeStruct((batch, nf), jnp.float32)),
        compiler_params=pltpu.CompilerParams(needs_layout_passes=False),
        scratch_types=[
            pltpu.VMEM((batch,), jnp.int32),
            pltpu.VMEM((batch,), jnp.int32),
            pltpu.VMEM((2, nf, STAGE), jnp.float32),
            pltpu.VMEM((RING, nf), jnp.float32),
            pltpu.VMEM((64 * 64,), jnp.float32),
            pltpu.SemaphoreType.DMA,
            pltpu.SemaphoreType.DMA,
        ],
    )
    def sc_kernel(u_hbm, i_hbm, uvT_hbm, ivT_hbm, utail_hbm, itail_hbm,
                  uout_hbm, iout_hbm,
                  idx_v, hits_v, blk, ring, tail_v, semb, semo):
        wid = lax.axis_index("s") * 2 + lax.axis_index("c")
        lane_iota = lax.broadcasted_iota(jnp.int32, (LANES,), 0)
        zeros16 = jnp.zeros((LANES,), jnp.int32)

        def extract(vec, k):
            return jnp.sum(jnp.where(lane_iota == k, vec, 0))

        def lag_wait(out_hbm):
            pltpu.make_async_copy(ring.at[pl.ds(0, 1), :],
                                  out_hbm.at[pl.ds(0, 1), :], semo).wait()

        def run_phase(src_idx_hbm, table_hbm, tail_hbm, out_hbm,
                      n_full, last_dma, tail, n_stages, max_iters, rc0):
            pltpu.sync_copy(src_idx_hbm, idx_v)
            if tail:
                pltpu.sync_copy(tail_hbm, tail_v.at[pl.ds(0, tail * nf)])

            def comp(g, cnt):
                v = idx_v[pl.ds(g * LANES, LANES)]
                mine = ((v >> SHIFT) & (NW - 1)) == wid
                mi = mine.astype(jnp.int32)
                off = plsc.cumsum(mi) - 1
                plsc.store_scatter(hits_v, [cnt + off],
                                   g * LANES + lane_iota, mask=mine)
                return cnt + jnp.sum(mi)

            nhits = lax.fori_loop(0, batch // LANES, comp, 0)
            nhg = (nhits + LANES - 1) // LANES
            last_stage = n_stages - 1

            def fire(s, p):
                @pl.when(s < n_full)
                def _():
                    pltpu.async_copy(
                        table_hbm.at[:, pl.ds(s * STAGE, STAGE)],
                        blk.at[p], semb)
                if last_dma:
                    @pl.when(s == n_full)
                    def _():
                        pltpu.async_copy(
                            table_hbm.at[:, pl.ds(n_full * STAGE, last_dma)],
                            blk.at[p, :, pl.ds(0, last_dma)], semb)

            def wait_blk(s):
                @pl.when(s < n_full)
                def _():
                    pltpu.make_async_copy(
                        table_hbm.at[:, pl.ds(0, STAGE)],
                        blk.at[0], semb).wait()
                if last_dma:
                    @pl.when(s == n_full)
                    def _():
                        pltpu.make_async_copy(
                            table_hbm.at[:, pl.ds(0, last_dma)],
                            blk.at[0, :, pl.ds(0, last_dma)], semb).wait()

            def backfill(p, col0):
                def one_col(c, _):
                    for t in range(nf // LANES):
                        flat = (t * LANES + lane_iota) * tail + c
                        vals = plsc.load_gather(tail_v, [flat])
                        plsc.store_scatter(
                            blk, [zeros16 + p, t * LANES + lane_iota,
                                  zeros16 + (col0 + c)], vals)
                    return _
                lax.fori_loop(0, tail, one_col, None)

            fire(wid, 0)

            def stage_body(k, rcnt):
                s = wid + k * NW

                def process(rc):
                    p = k & 1
                    nxt = s + NW

                    @pl.when(nxt < n_stages)
                    def _():
                        fire(nxt, (k + 1) & 1)

                    wait_blk(s)
                    if tail:
                        @pl.when(s == last_stage)
                        def _():
                            backfill(p, last_dma)

                    base = s * STAGE

                    def hit_group(g, rc):
                        hb = hits_v[pl.ds(g * LANES, LANES)] & (batch - 1)
                        hu = plsc.load_gather(idx_v, [hb])
                        valid = (g * LANES + lane_iota) < nhits
                        smask = ((hu >> SHIFT) == s) & valid
                        bits = jnp.sum(jnp.where(smask,
                                                 jnp.int32(1) << lane_iota,
                                                 0))

                        def do_lanes(rc):
                            for kk in range(LANES):
                                def work(rc, kk=kk):
                                    b = extract(hb, kk)
                                    col = extract(hu, kk) - base

                                    @pl.when(rc >= RING)
                                    def _():
                                        lag_wait(out_hbm)
                                    slot = rc % RING
                                    for t in range(nf // LANES):
                                        vals = plsc.load_gather(
                                            blk,
                                            [zeros16 + p,
                                             t * LANES + lane_iota,
                                             zeros16 + col])
                                        ring[slot,
                                             pl.ds(t * LANES, LANES)] = vals
                                    pltpu.async_copy(
                                        ring.at[pl.ds(slot, 1), :],
                                        out_hbm.at[pl.ds(b, 1), :], semo)
                                    return rc + 1

                                mk = ((bits >> kk) & 1) > 0
                                rc = lax.cond(mk, work, lambda r: r, rc)
                            return rc

                        return lax.cond(bits > 0, do_lanes,
                                        lambda r: r, rc)

                    return lax.fori_loop(0, nhg, hit_group, rc)

                return lax.cond(s < n_stages, process,
                                lambda r: r, rcnt)

            return lax.fori_loop(0, max_iters, stage_body, rc0)

        rc = run_phase(u_hbm, uvT_hbm, utail_hbm, uout_hbm,
                       uf, ul, utl, us, max_it_u, 0)
        rc = run_phase(i_hbm, ivT_hbm, itail_hbm, iout_hbm,
                       itf, il, itl, is_, max_it_i, rc)

        def drain(j, _):
            @pl.when(j < jnp.minimum(rc, RING))
            def _():
                lag_wait(uout_hbm)
            return _

        lax.fori_loop(0, RING, drain, None)

    return sc_kernel


def _dot_kernel(batch, nf):
    blk = 1024

    def body(a_ref, b_ref, o_ref):
        o_ref[...] = jnp.sum(a_ref[...] * b_ref[...], axis=1)

    return pl.pallas_call(
        body,
        grid=(batch // blk,),
        in_specs=[pl.BlockSpec((blk, nf), lambda g: (g, 0)),
                  pl.BlockSpec((blk, nf), lambda g: (g, 0))],
        out_specs=pl.BlockSpec((blk,), lambda g: (g,)),
        out_shape=jax.ShapeDtypeStruct((batch,), jnp.float32),
    )


def kernel(u, i, user_vec, item_vec):
    batch = u.shape[0]
    n_user, nf = user_vec.shape
    n_item = item_vec.shape[0]
    uvT = user_vec.T
    ivT = item_vec.T
    utail = uvT[:, (n_user // 128) * 128:].reshape(-1)
    itail = ivT[:, (n_item // 128) * 128:].reshape(-1)
    scan = _scan_kernel(batch, nf, n_user, n_item)
    uvg, ivg = scan(u, i, uvT, ivT, utail, itail)
    return _dot_kernel(batch, nf)(uvg, ivg)


# speedup vs baseline: 1.3061x; 1.2973x over previous
import functools

import jax
import jax.numpy as jnp
from jax import lax
from jax.experimental import pallas as pl
from jax.experimental.pallas import tpu as pltpu
from jax.experimental.pallas import tpu_sc as plsc

LANES = 16
CHUNK = 128
GROUPS = CHUNK // LANES


def _sc_dot_kernel(batch, n_factors, n_workers, nc):
    bpw = batch // n_workers
    nch = bpw // CHUNK

    mesh = plsc.VectorSubcoreMesh(core_axis_name="c", subcore_axis_name="s")

    @functools.partial(
        pl.kernel,
        mesh=mesh,
        out_type=jax.ShapeDtypeStruct((batch,), jnp.float32),
        compiler_params=pltpu.CompilerParams(
            needs_layout_passes=False, skip_device_barrier=True),
        scratch_types=[
            pltpu.VMEM((bpw,), jnp.int32),
            pltpu.VMEM((bpw,), jnp.int32),
            pltpu.VMEM((CHUNK, 64), jnp.float32),
            pltpu.VMEM((CHUNK, 64), jnp.float32),
            pltpu.VMEM((CHUNK, 64), jnp.float32),
            pltpu.VMEM((CHUNK, 64), jnp.float32),
            pltpu.VMEM((bpw,), jnp.float32),
            pltpu.SemaphoreType.DMA,
            pltpu.SemaphoreType.DMA,
        ],
    )
    def sc_kernel(u_hbm, i_hbm, uvec_hbm, ivec_hbm, out_hbm,
                  uidx_v, iidx_v, ubuf0, ibuf0, ubuf1, ibuf1, out_v,
                  sem0, sem1):
        wid = lax.axis_index("s") * nc + lax.axis_index("c")
        base = wid * bpw
        lane_iota = lax.broadcasted_iota(jnp.int32, (LANES,), 0)

        pltpu.sync_copy(u_hbm.at[pl.ds(base, bpw)], uidx_v)
        pltpu.sync_copy(i_hbm.at[pl.ds(base, bpw)], iidx_v)

        ubufs = (ubuf0, ubuf1)
        ibufs = (ibuf0, ibuf1)
        sems = (sem0, sem1)

        def fire(c, ub, ib, sem):
            def group(g, _):
                uv = uidx_v[pl.ds(c * CHUNK + g * LANES, LANES)]
                iv = iidx_v[pl.ds(c * CHUNK + g * LANES, LANES)]
                for k in range(LANES):
                    m = lane_iota == k
                    ru = jnp.sum(jnp.where(m, uv, 0))
                    ri = jnp.sum(jnp.where(m, iv, 0))
                    r = g * LANES + k
                    pltpu.async_copy(uvec_hbm.at[pl.ds(ru, 1), :],
                                     ub.at[pl.ds(r, 1), :], sem)
                    pltpu.async_copy(ivec_hbm.at[pl.ds(ri, 1), :],
                                     ib.at[pl.ds(r, 1), :], sem)
                return _
            lax.fori_loop(0, GROUPS, group, None)

        def drain(ub, ib, sem):
            pltpu.make_async_copy(uvec_hbm.at[pl.ds(0, CHUNK), :],
                                  ub, sem).wait()
            pltpu.make_async_copy(ivec_hbm.at[pl.ds(0, CHUNK), :],
                                  ib, sem).wait()

        def compute(c, ub, ib):
            def group(g, _):
                rows = g * LANES + lane_iota
                acc = jnp.zeros((LANES,), jnp.float32)
                for f in range(n_factors):
                    cols = jnp.full((LANES,), f, jnp.int32)
                    uvals = plsc.load_gather(ub, [rows, cols])
                    ivals = plsc.load_gather(ib, [rows, cols])
                    acc = acc + uvals * ivals
                out_v[pl.ds(c * CHUNK + g * LANES, LANES)] = acc
                return _
            lax.fori_loop(0, GROUPS, group, None)

        fire(0, ubufs[0], ibufs[0], sems[0])
        for c in range(nch):
            p = c % 2
            if c + 1 < nch:
                fire(c + 1, ubufs[1 - p], ibufs[1 - p], sems[1 - p])
            drain(ubufs[p], ibufs[p], sems[p])
            compute(c, ubufs[p], ibufs[p])

        pltpu.sync_copy(out_v, out_hbm.at[pl.ds(base, bpw)])

    return sc_kernel


def kernel(u, i, user_vec, item_vec):
    batch = u.shape[0]
    n_factors = user_vec.shape[1]
    info = plsc.get_sparse_core_info()
    nc, ns = info.num_cores, info.num_subcores
    n_workers = nc * ns
    fn = _sc_dot_kernel(batch, n_factors, n_workers, nc)
    return fn(u, i, user_vec, item_vec)
